# baseline (speedup 1.0000x reference)
import jax
import jax.numpy as jnp
from jax import lax
from jax.experimental import pallas as pl
from jax.experimental.pallas import tpu as pltpu

_CH = 40
_NCHUNK = 10
_RB = _CH * _NCHUNK


def _gcn_body(a_hbm, x_hbm, w_ref, b_ref, o_hbm, abuf, xbuf, sup, obuf,
              asem, xsem, osem):
    n = a_hbm.shape[0]
    nrounds = n // _RB

    def start_block(r, blk):
        for c in range(_NCHUNK):
            pltpu.make_async_copy(
                a_hbm.at[pl.ds(r * _RB + c * _CH, _CH), :],
                abuf.at[blk, pl.ds(c * _CH, _CH), :],
                asem.at[blk],
            ).start()

    def wait_block(r, blk):
        pltpu.make_async_copy(
            a_hbm.at[pl.ds(r * _RB, _RB), :], abuf.at[blk], asem.at[blk]
        ).wait()

    pltpu.make_async_copy(x_hbm, xbuf, xsem).start()
    start_block(0, 0)
    start_block(1, 1)
    pltpu.make_async_copy(x_hbm, xbuf, xsem).wait()
    sup[...] = jnp.dot(
        xbuf[...], w_ref[...], preferred_element_type=jnp.float32
    ).astype(jnp.bfloat16)

    def round_body(r, carry):
        blk = lax.rem(r, 2)
        wait_block(r, blk)

        @pl.when(r >= 2)
        def _():
            pltpu.make_async_copy(
                obuf.at[blk],
                o_hbm.at[pl.ds((r - 2) * _RB, _RB), :],
                osem.at[blk],
            ).wait()

        obuf[blk] = (
            jnp.dot(
                abuf[blk].astype(jnp.bfloat16),
                sup[...],
                preferred_element_type=jnp.float32,
            )
            + b_ref[...]
        )
        pltpu.make_async_copy(
            obuf.at[blk], o_hbm.at[pl.ds(r * _RB, _RB), :], osem.at[blk]
        ).start()

        @pl.when(r + 2 < nrounds)
        def _():
            start_block(r + 2, blk)

        return carry

    lax.fori_loop(0, nrounds, round_body, 0, unroll=False)

    pltpu.make_async_copy(
        obuf.at[lax.rem(nrounds - 2, 2)],
        o_hbm.at[pl.ds((nrounds - 2) * _RB, _RB), :],
        osem.at[lax.rem(nrounds - 2, 2)],
    ).wait()
    pltpu.make_async_copy(
        obuf.at[lax.rem(nrounds - 1, 2)],
        o_hbm.at[pl.ds((nrounds - 1) * _RB, _RB), :],
        osem.at[lax.rem(nrounds - 1, 2)],
    ).wait()


def kernel(x_feature, adjacency_matrix, weight, bias):
    n, in_dim = x_feature.shape
    out_dim = weight.shape[1]
    bias2 = bias.reshape(1, out_dim)
    return pl.pallas_call(
        _gcn_body,
        in_specs=[
            pl.BlockSpec(memory_space=pl.ANY),
            pl.BlockSpec(memory_space=pl.ANY),
            pl.BlockSpec(memory_space=pltpu.VMEM),
            pl.BlockSpec(memory_space=pltpu.VMEM),
        ],
        out_specs=pl.BlockSpec(memory_space=pl.ANY),
        out_shape=jax.ShapeDtypeStruct((n, out_dim), jnp.float32),
        scratch_shapes=[
            pltpu.VMEM((2, _RB, n), jnp.float32),
            pltpu.VMEM((n, in_dim), jnp.float32),
            pltpu.VMEM((n, out_dim), jnp.bfloat16),
            pltpu.VMEM((2, _RB, out_dim), jnp.float32),
            pltpu.SemaphoreType.DMA((2,)),
            pltpu.SemaphoreType.DMA,
            pltpu.SemaphoreType.DMA((2,)),
        ],
    )(adjacency_matrix, x_feature, weight, bias2)


# speedup vs baseline: 1.0044x; 1.0044x over previous
import jax
import jax.numpy as jnp
from jax import lax
from jax.experimental import pallas as pl
from jax.experimental.pallas import tpu as pltpu

_RB = 200
_NBLK = 4


def _gcn_body(a_hbm, x_hbm, w_ref, b_ref, o_hbm, abuf, xbuf, sup, obuf,
              asem, xsem, osem):
    n = a_hbm.shape[0]
    nrounds = n // _RB

    def block_copy(r, blk):
        return pltpu.make_async_copy(
            a_hbm.at[pl.ds(r * _RB, _RB), :], abuf.at[blk], asem.at[blk]
        )

    pltpu.make_async_copy(x_hbm, xbuf, xsem).start()
    for b in range(_NBLK):
        block_copy(b, b).start()
    pltpu.make_async_copy(x_hbm, xbuf, xsem).wait()
    sup[...] = jnp.dot(
        xbuf[...], w_ref[...], preferred_element_type=jnp.float32
    ).astype(jnp.bfloat16)

    def round_body(r, carry):
        blk = lax.rem(r, _NBLK)
        ob = lax.rem(r, 2)
        block_copy(r, blk).wait()

        @pl.when(r >= 2)
        def _():
            pltpu.make_async_copy(
                obuf.at[ob],
                o_hbm.at[pl.ds((r - 2) * _RB, _RB), :],
                osem.at[ob],
            ).wait()

        obuf[ob] = (
            jnp.dot(
                abuf[blk].astype(jnp.bfloat16),
                sup[...],
                preferred_element_type=jnp.float32,
            )
            + b_ref[...]
        )
        pltpu.make_async_copy(
            obuf.at[ob], o_hbm.at[pl.ds(r * _RB, _RB), :], osem.at[ob]
        ).start()

        @pl.when(r + _NBLK < nrounds)
        def _():
            block_copy(r + _NBLK, blk).start()

        return carry

    lax.fori_loop(0, nrounds, round_body, 0, unroll=False)

    pltpu.make_async_copy(
        obuf.at[lax.rem(nrounds - 2, 2)],
        o_hbm.at[pl.ds((nrounds - 2) * _RB, _RB), :],
        osem.at[lax.rem(nrounds - 2, 2)],
    ).wait()
    pltpu.make_async_copy(
        obuf.at[lax.rem(nrounds - 1, 2)],
        o_hbm.at[pl.ds((nrounds - 1) * _RB, _RB), :],
        osem.at[lax.rem(nrounds - 1, 2)],
    ).wait()


def kernel(x_feature, adjacency_matrix, weight, bias):
    n, in_dim = x_feature.shape
    out_dim = weight.shape[1]
    bias2 = bias.reshape(1, out_dim)
    return pl.pallas_call(
        _gcn_body,
        in_specs=[
            pl.BlockSpec(memory_space=pl.ANY),
            pl.BlockSpec(memory_space=pl.ANY),
            pl.BlockSpec(memory_space=pltpu.VMEM),
            pl.BlockSpec(memory_space=pltpu.VMEM),
        ],
        out_specs=pl.BlockSpec(memory_space=pl.ANY),
        out_shape=jax.ShapeDtypeStruct((n, out_dim), jnp.float32),
        scratch_shapes=[
            pltpu.VMEM((_NBLK, _RB, n), jnp.float32),
            pltpu.VMEM((n, in_dim), jnp.float32),
            pltpu.VMEM((n, out_dim), jnp.bfloat16),
            pltpu.VMEM((2, _RB, out_dim), jnp.float32),
            pltpu.SemaphoreType.DMA((_NBLK,)),
            pltpu.SemaphoreType.DMA,
            pltpu.SemaphoreType.DMA((2,)),
        ],
    )(adjacency_matrix, x_feature, weight, bias2)


# speedup vs baseline: 1.0229x; 1.0184x over previous
import jax
import jax.numpy as jnp
from jax.experimental import pallas as pl
from jax.experimental.pallas import tpu as pltpu

_BM = 400


def _gcn_body(x_ref, w_ref, a_ref, b_ref, o_ref, s_ref):
    @pl.when(pl.program_id(0) == 0)
    def _():
        s_ref[...] = jnp.dot(
            x_ref[...], w_ref[...], preferred_element_type=jnp.float32
        ).astype(jnp.bfloat16)

    acc = jnp.dot(
        a_ref[...].astype(jnp.bfloat16),
        s_ref[...],
        preferred_element_type=jnp.float32,
    )
    o_ref[...] = acc + b_ref[...]


def kernel(x_feature, adjacency_matrix, weight, bias):
    n, in_dim = x_feature.shape
    out_dim = weight.shape[1]
    bias2 = bias.reshape(1, out_dim)
    return pl.pallas_call(
        _gcn_body,
        grid=(pl.cdiv(n, _BM),),
        in_specs=[
            pl.BlockSpec((n, in_dim), lambda i: (0, 0)),
            pl.BlockSpec((in_dim, out_dim), lambda i: (0, 0)),
            pl.BlockSpec((_BM, n), lambda i: (i, 0)),
            pl.BlockSpec((1, out_dim), lambda i: (0, 0)),
        ],
        out_specs=pl.BlockSpec((_BM, out_dim), lambda i: (i, 0)),
        out_shape=jax.ShapeDtypeStruct((n, out_dim), jnp.float32),
        scratch_shapes=[pltpu.VMEM((n, out_dim), jnp.bfloat16)],
        compiler_params=pltpu.CompilerParams(
            dimension_semantics=("arbitrary",),
        ),
    )(x_feature, weight, adjacency_matrix, bias2)
